# baseline (speedup 1.0000x reference)
import functools

import jax
import jax.numpy as jnp
from jax import lax
from jax.experimental import pallas as pl
from jax.experimental.pallas import tpu as pltpu
from jax.experimental.pallas import tpu_sc as plsc

_TILE_T = 2048


def _mm_body(kT_ref, x_ref, out_ref):
    out_ref[...] = lax.dot_general(
        kT_ref[...], x_ref[...],
        dimension_numbers=(((1,), (1,)), ((), ())),
        preferred_element_type=jnp.float32)


def _router_logits_et(x, kT):
    T, D = x.shape
    E = kT.shape[0]
    return pl.pallas_call(
        _mm_body,
        grid=(T // _TILE_T,),
        in_specs=[
            pl.BlockSpec((E, D), lambda i: (0, 0)),
            pl.BlockSpec((_TILE_T, D), lambda i: (i, 0)),
        ],
        out_specs=pl.BlockSpec((E, _TILE_T), lambda i: (0, i)),
        out_shape=jax.ShapeDtypeStruct((E, T), jnp.float32),
    )(kT, x)


def _sc_topk2(logits_et):
    E, T = logits_et.shape
    info = plsc.get_sparse_core_info()
    nc, ns, L = info.num_cores, info.num_subcores, info.num_lanes
    nw = nc * ns
    tpw = T // nw
    n_groups = tpw // L

    @functools.partial(
        pl.kernel,
        out_type=[
            jax.ShapeDtypeStruct((2 * T,), jnp.float32),
            jax.ShapeDtypeStruct((2 * T,), jnp.int32),
        ],
        mesh=plsc.VectorSubcoreMesh(core_axis_name="c", subcore_axis_name="s"),
        compiler_params=pltpu.CompilerParams(needs_layout_passes=False,
                                             use_tc_tiling_on_sc=True,
                                             skip_device_barrier=True),
        scratch_types=[
            pltpu.VMEM((E, tpw), jnp.float32),
            pltpu.VMEM((2 * tpw,), jnp.float32),
            pltpu.VMEM((2 * tpw,), jnp.int32),
        ],
    )
    def topk_kernel(logits_hbm, w_hbm, ids_hbm, logits_v, w_v, ids_v):
        wid = lax.axis_index("s") * nc + lax.axis_index("c")
        pltpu.sync_copy(logits_hbm.at[:, pl.ds(wid * tpw, tpw)], logits_v)
        iota = lax.iota(jnp.int32, L)

        def group(g, carry):
            neg = jnp.full((L,), -jnp.inf, jnp.float32)
            m1, m2 = neg, neg
            i1 = jnp.zeros((L,), jnp.int32)
            i2 = jnp.zeros((L,), jnp.int32)
            for e in range(E):
                es = jnp.full((L,), e, jnp.int32)
                v = logits_v[e, pl.ds(g * L, L)]
                gt1 = v > m1
                gt2 = v > m2
                m2 = jnp.where(gt1, m1, jnp.where(gt2, v, m2))
                i2 = jnp.where(gt1, i1, jnp.where(gt2, es, i2))
                m1 = jnp.where(gt1, v, m1)
                i1 = jnp.where(gt1, es, i1)
            ed = jnp.exp(m2 - m1)
            w1 = 1.0 / (1.0 + ed)
            w2 = 1.0 - w1
            ob = (g // 8) * 256 + (g % 8) * 16
            plsc.store_scatter(w_v, [ob + iota], w1)
            plsc.store_scatter(w_v, [ob + 128 + iota], w2)
            plsc.store_scatter(ids_v, [ob + iota], i1)
            plsc.store_scatter(ids_v, [ob + 128 + iota], i2)
            return carry

        lax.fori_loop(0, n_groups, group, 0)
        pltpu.sync_copy(w_v, w_hbm.at[pl.ds(wid * 2 * tpw, 2 * tpw)])
        pltpu.sync_copy(ids_v, ids_hbm.at[pl.ds(wid * 2 * tpw, 2 * tpw)])

    return topk_kernel(logits_et)


def kernel(x, kernel_DE):
    T = x.shape[0]
    logits_et = _router_logits_et(x, kernel_DE.T)
    w_flat, ids_flat = _sc_topk2(logits_et)
    w = w_flat.reshape(T // 128, 2, 128).transpose(0, 2, 1).reshape(T, 2)
    ids = ids_flat.reshape(T // 128, 2, 128).transpose(0, 2, 1).reshape(T, 2)
    return (w, ids)


# speedup vs baseline: 1.0414x; 1.0414x over previous
import functools

import jax
import jax.numpy as jnp
from jax import lax
from jax.experimental import pallas as pl
from jax.experimental.pallas import tpu as pltpu
from jax.experimental.pallas import tpu_sc as plsc

_TILE_T = 1024


def _mm_body(kT_ref, x_ref, out_ref):
    out_ref[...] = lax.dot_general(
        kT_ref[...], x_ref[...],
        dimension_numbers=(((1,), (1,)), ((), ())),
        preferred_element_type=jnp.float32)


def _router_logits_et(x, kT):
    T, D = x.shape
    E = kT.shape[0]
    return pl.pallas_call(
        _mm_body,
        grid=(T // _TILE_T,),
        in_specs=[
            pl.BlockSpec((E, D), lambda i: (0, 0)),
            pl.BlockSpec((_TILE_T, D), lambda i: (i, 0)),
        ],
        out_specs=pl.BlockSpec((E, _TILE_T), lambda i: (0, i)),
        out_shape=jax.ShapeDtypeStruct((E, T), jnp.float32),
    )(kT, x)


def _sc_topk2(logits_et):
    E, T = logits_et.shape
    info = plsc.get_sparse_core_info()
    nc, ns, L = info.num_cores, info.num_subcores, info.num_lanes
    nw = nc * ns
    tpw = T // nw
    n_groups = tpw // L

    @functools.partial(
        pl.kernel,
        out_type=[
            jax.ShapeDtypeStruct((2 * T,), jnp.float32),
            jax.ShapeDtypeStruct((2 * T,), jnp.int32),
        ],
        mesh=plsc.VectorSubcoreMesh(core_axis_name="c", subcore_axis_name="s"),
        compiler_params=pltpu.CompilerParams(needs_layout_passes=False,
                                             use_tc_tiling_on_sc=True,
                                             skip_device_barrier=True),
        scratch_types=[
            pltpu.VMEM((E, tpw), jnp.float32),
            pltpu.VMEM((2 * tpw,), jnp.float32),
            pltpu.VMEM((2 * tpw,), jnp.int32),
        ],
    )
    def topk_kernel(logits_hbm, w_hbm, ids_hbm, logits_v, w_v, ids_v):
        wid = lax.axis_index("s") * nc + lax.axis_index("c")
        pltpu.sync_copy(logits_hbm.at[:, pl.ds(wid * tpw, tpw)], logits_v)
        iota = lax.iota(jnp.int32, L)

        def group(g, carry):
            neg = jnp.full((L,), -jnp.inf, jnp.float32)
            m1, m2 = neg, neg
            i1 = jnp.zeros((L,), jnp.int32)
            i2 = jnp.zeros((L,), jnp.int32)
            for e in range(E):
                es = jnp.full((L,), e, jnp.int32)
                v = logits_v[e, pl.ds(g * L, L)]
                gt1 = v > m1
                gt2 = v > m2
                m2 = jnp.where(gt1, m1, jnp.where(gt2, v, m2))
                i2 = jnp.where(gt1, i1, jnp.where(gt2, es, i2))
                m1 = jnp.where(gt1, v, m1)
                i1 = jnp.where(gt1, es, i1)
            ed = jnp.exp(m2 - m1)
            w1 = 1.0 / (1.0 + ed)
            w2 = 1.0 - w1
            ob = (g // 8) * 256 + (g % 8) * 16
            plsc.store_scatter(w_v, [ob + iota], w1)
            plsc.store_scatter(w_v, [ob + 128 + iota], w2)
            plsc.store_scatter(ids_v, [ob + iota], i1)
            plsc.store_scatter(ids_v, [ob + 128 + iota], i2)
            return carry

        lax.fori_loop(0, n_groups, group, 0)
        pltpu.sync_copy(w_v, w_hbm.at[pl.ds(wid * 2 * tpw, 2 * tpw)])
        pltpu.sync_copy(ids_v, ids_hbm.at[pl.ds(wid * 2 * tpw, 2 * tpw)])

    return topk_kernel(logits_et)


def kernel(x, kernel_DE):
    T = x.shape[0]
    logits_et = _router_logits_et(x, kernel_DE.T)
    w_flat, ids_flat = _sc_topk2(logits_et)
    w = w_flat.reshape(T // 128, 2, 128).transpose(0, 2, 1).reshape(T, 2)
    ids = ids_flat.reshape(T // 128, 2, 128).transpose(0, 2, 1).reshape(T, 2)
    return (w, ids)
